# baseline (speedup 1.0000x reference)
import functools

import jax
import jax.numpy as jnp
from jax import lax
from jax.experimental import pallas as pl
from jax.experimental.pallas import tpu as pltpu
from jax.experimental.pallas import tpu_sc as plsc

EMBED = 64
NUM_WORKERS = 32
CHUNK = 128


def _gather_body(table_hbm, idx_hbm, out_hbm, idx_v, rows_v, sem):
    c = lax.axis_index("c")
    s = lax.axis_index("s")
    wid = s * 2 + c
    nchunk = idx_v.shape[0]
    pltpu.sync_copy(idx_hbm.at[wid], idx_v)

    def body(j, carry):
        pltpu.async_copy(table_hbm.at[idx_v.at[j]], rows_v, sem).wait()
        pltpu.sync_copy(rows_v, out_hbm.at[wid, j])
        return carry

    lax.fori_loop(0, nchunk, body, 0)


def _make_gather(nchunk):
    return functools.partial(
        pl.kernel,
        out_type=jax.ShapeDtypeStruct((NUM_WORKERS, nchunk, CHUNK, EMBED), jnp.float32),
        mesh=plsc.VectorSubcoreMesh(core_axis_name="c", subcore_axis_name="s"),
        scratch_types=[
            pltpu.VMEM((nchunk, CHUNK), jnp.int32),
            pltpu.VMEM((CHUNK, EMBED), jnp.float32),
            pltpu.SemaphoreType.DMA,
        ],
        compiler_params=pltpu.CompilerParams(use_tc_tiling_on_sc=False),
    )(_gather_body)


def kernel(word_input, character_input, word_embed):
    batch, seq = word_input.shape
    total = batch * seq
    nchunk = total // (NUM_WORKERS * CHUNK)
    idx = word_input.reshape(NUM_WORKERS, nchunk, CHUNK)
    out = _make_gather(nchunk)(word_embed, idx)
    return out.reshape(batch, seq, EMBED)


# speedup vs baseline: 1.0462x; 1.0462x over previous
import functools

import jax
import jax.numpy as jnp
from jax import lax
from jax.experimental import pallas as pl
from jax.experimental.pallas import tpu as pltpu
from jax.experimental.pallas import tpu_sc as plsc

EMBED = 64
NUM_WORKERS = 32
CHUNK = 128
NBUF = 10


def _gather_body(table_hbm, idx_hbm, out_hbm, idx_v, rows, gsem, osem):
    c = lax.axis_index("c")
    s = lax.axis_index("s")
    wid = s * 2 + c
    nchunk = idx_v.shape[0]
    nround = nchunk // NBUF
    pltpu.sync_copy(idx_hbm.at[wid], idx_v)

    for b in range(NBUF):
        pltpu.async_copy(table_hbm.at[idx_v.at[b]], rows.at[b], gsem.at[b])

    def round_body(g, carry):
        for b in range(NBUF):
            j = g * NBUF + b
            pltpu.make_async_copy(table_hbm.at[idx_v.at[b]], rows.at[b],
                                  gsem.at[b]).wait()
            pltpu.async_copy(rows.at[b], out_hbm.at[wid, j], osem.at[b])

            @pl.when(g < nround - 1)
            def _():
                pltpu.make_async_copy(rows.at[b], out_hbm.at[wid, j],
                                      osem.at[b]).wait()
                pltpu.async_copy(table_hbm.at[idx_v.at[j + NBUF]], rows.at[b],
                                 gsem.at[b])
        return carry

    lax.fori_loop(0, nround, round_body, 0)

    for b in range(NBUF):
        pltpu.make_async_copy(rows.at[b], out_hbm.at[wid, 0], osem.at[b]).wait()


def _make_gather(nchunk):
    return functools.partial(
        pl.kernel,
        out_type=jax.ShapeDtypeStruct((NUM_WORKERS, nchunk, CHUNK, EMBED), jnp.float32),
        mesh=plsc.VectorSubcoreMesh(core_axis_name="c", subcore_axis_name="s"),
        scratch_types=[
            pltpu.VMEM((nchunk, CHUNK), jnp.int32),
            pltpu.VMEM((NBUF, CHUNK, EMBED), jnp.float32),
            pltpu.SemaphoreType.DMA((NBUF,)),
            pltpu.SemaphoreType.DMA((NBUF,)),
        ],
        compiler_params=pltpu.CompilerParams(use_tc_tiling_on_sc=False),
    )(_gather_body)


def kernel(word_input, character_input, word_embed):
    batch, seq = word_input.shape
    total = batch * seq
    nchunk = total // (NUM_WORKERS * CHUNK)
    idx = word_input.reshape(NUM_WORKERS, nchunk, CHUNK)
    out = _make_gather(nchunk)(word_embed, idx)
    return out.reshape(batch, seq, EMBED)


# speedup vs baseline: 1.0552x; 1.0086x over previous
import functools

import jax
import jax.numpy as jnp
from jax import lax
from jax.experimental import pallas as pl
from jax.experimental.pallas import tpu as pltpu
from jax.experimental.pallas import tpu_sc as plsc

EMBED = 64
NUM_WORKERS = 32
CHUNK = 128
BLOCKS = 8
MAX_ROWS = 7


def _gather_body(table_hbm, idxT_hbm, out_hbm, idx_v, rows, gsem, osem):
    c = lax.axis_index("c")
    s = lax.axis_index("s")
    wid = s * 2 + c
    seq = idxT_hbm.shape[0]
    n_extra = seq - (seq // NUM_WORKERS) * NUM_WORKERS
    base_rows = seq // NUM_WORKERS
    start = jnp.where(wid < n_extra, wid * (base_rows + 1),
                      n_extra * (base_rows + 1) + (wid - n_extra) * base_rows)
    n_rows = jnp.where(wid < n_extra, base_rows + 1, base_rows)

    pltpu.sync_copy(idxT_hbm.at[pl.ds(start, base_rows)],
                    idx_v.at[pl.ds(0, base_rows)])

    @pl.when(n_rows > base_rows)
    def _():
        pltpu.sync_copy(idxT_hbm.at[pl.ds(start + base_rows, 1)],
                        idx_v.at[pl.ds(base_rows, 1)])

    for b in range(BLOCKS):
        pltpu.async_copy(table_hbm.at[idx_v.at[0, pl.ds(b * CHUNK, CHUNK)]],
                         rows.at[b], gsem.at[b])

    def row_body(si, carry):
        row = start + si
        for b in range(BLOCKS):
            pltpu.make_async_copy(
                table_hbm.at[idx_v.at[0, pl.ds(b * CHUNK, CHUNK)]],
                rows.at[b], gsem.at[b]).wait()
            pltpu.async_copy(rows.at[b], out_hbm.at[row, pl.ds(b * CHUNK, CHUNK)],
                             osem.at[b])

            @pl.when(si < n_rows - 1)
            def _():
                pltpu.make_async_copy(
                    rows.at[b], out_hbm.at[row, pl.ds(b * CHUNK, CHUNK)],
                    osem.at[b]).wait()
                pltpu.async_copy(
                    table_hbm.at[idx_v.at[si + 1, pl.ds(b * CHUNK, CHUNK)]],
                    rows.at[b], gsem.at[b])
        return carry

    lax.fori_loop(0, n_rows, row_body, 0)

    for b in range(BLOCKS):
        pltpu.make_async_copy(rows.at[b], out_hbm.at[0, pl.ds(0, CHUNK)],
                              osem.at[b]).wait()


def _make_gather(seq, batch):
    return functools.partial(
        pl.kernel,
        out_type=jax.ShapeDtypeStruct((seq, batch, EMBED), jnp.float32),
        mesh=plsc.VectorSubcoreMesh(core_axis_name="c", subcore_axis_name="s"),
        scratch_types=[
            pltpu.VMEM((MAX_ROWS, batch), jnp.int32),
            pltpu.VMEM((BLOCKS, CHUNK, EMBED), jnp.float32),
            pltpu.SemaphoreType.DMA((BLOCKS,)),
            pltpu.SemaphoreType.DMA((BLOCKS,)),
        ],
        compiler_params=pltpu.CompilerParams(use_tc_tiling_on_sc=False),
    )(_gather_body)


def kernel(word_input, character_input, word_embed):
    batch, seq = word_input.shape
    idxT = word_input.T
    out = _make_gather(seq, batch)(word_embed, idxT)
    return jnp.transpose(out, (1, 0, 2))
